# baseline (speedup 1.0000x reference)
import jax
import jax.numpy as jnp
from jax.experimental import pallas as pl
from jax.experimental.pallas import tpu as pltpu

D = 768
SEQ = 8192
B = 4
T = 256
NT = SEQ // T
EPS = 1e-8
THR = 0.5
N_TGT = 6.0
RATIO_W = 0.03
PPW = 8


def _body(tok_ref, w_ref, sk_ref, down_hbm, pp_hbm, aux_ref,
          down_ref, pp_ref, carry_ref, base_ref, sumg_ref, sem1, sem2):
    b = pl.program_id(0)
    t = pl.program_id(1)

    x = tok_ref[0]
    qk = jnp.dot(x, w_ref[...], preferred_element_type=jnp.float32)
    q = qk[:, :D]
    k = qk[:, D:]

    carry = jnp.where(t == 0, sk_ref[...], carry_ref[...])
    kprev = jnp.concatenate([carry, k[:-1, :]], axis=0)
    carry_ref[...] = k[T - 1:T, :]

    dot = jnp.sum(q * kprev, axis=1, keepdims=True)
    qn = jnp.sqrt(jnp.sum(q * q, axis=1, keepdims=True))
    kn = jnp.sqrt(jnp.sum(kprev * kprev, axis=1, keepdims=True))
    den = jnp.maximum(qn, EPS) * jnp.maximum(kn, EPS)
    cos = dot / den
    probs = (1.0 - cos) * 0.5

    pos0 = t * T
    sub_iota = jax.lax.broadcasted_iota(jnp.int32, (T, 1), 0)
    boundary = jnp.logical_or(probs > THR, (sub_iota + pos0) == 0)
    bf = boundary.astype(jnp.float32)

    row_i = jax.lax.broadcasted_iota(jnp.int32, (T, T), 0)
    col_i = jax.lax.broadcasted_iota(jnp.int32, (T, T), 1)
    tri = (col_i < row_i).astype(jnp.float32)
    ranks_f = jax.lax.dot_general(tri, bf, (((1,), (0,)), ((), ())),
                                  precision=jax.lax.Precision.HIGHEST)
    ranks = ranks_f.astype(jnp.int32)
    cnt = jnp.sum(bf).astype(jnp.int32)

    P = jnp.where(jnp.logical_and(boundary, col_i == ranks), 1.0, 0.0)

    posf = (sub_iota + pos0).astype(jnp.float32)
    extra = jnp.concatenate(
        [probs, posf, jnp.zeros((T, PPW - 2), jnp.float32)], axis=1)
    packed_tok = jax.lax.dot_general(P, x * probs, (((0,), (0,)), ((), ())),
                                     precision=jax.lax.Precision.HIGHEST)
    packed_extra = jax.lax.dot_general(P, extra, (((0,), (0,)), ((), ())),
                                       precision=jax.lax.Precision.HIGHEST)

    base_prev = jnp.where(t == 0, 0, base_ref[0])
    a8 = pl.multiple_of((base_prev // 8) * 8, 8)
    r = base_prev - a8

    @pl.when(jnp.logical_and(t == 0, b > 0))
    def _drain():
        pltpu.make_async_copy(down_ref, down_hbm.at[b - 1], sem1).wait()
        pltpu.make_async_copy(pp_ref, pp_hbm.at[b - 1], sem2).wait()

    down_ref[pl.ds(pos0, T), :] = jnp.zeros((T, D), jnp.float32)
    pp_ref[pl.ds(pos0, T), :] = jnp.zeros((T, PPW), jnp.float32)

    @pl.when(t == 0)
    def _zero_tail():
        pp_ref[pl.ds(SEQ, PPW), :] = jnp.zeros((PPW, PPW), jnp.float32)

    iota8 = jax.lax.broadcasted_iota(jnp.int32, (T + 8, 1), 0)
    ext_tok = jnp.concatenate(
        [packed_tok, jnp.zeros((8, D), jnp.float32)], axis=0)
    ext_extra = jnp.concatenate(
        [packed_extra, jnp.zeros((8, PPW), jnp.float32)], axis=0)
    sent = jnp.concatenate(
        [jnp.zeros((1, 1), jnp.float32),
         jnp.full((1, 1), float(SEQ), jnp.float32),
         jnp.zeros((1, PPW - 2), jnp.float32)], axis=1)
    is_last = t == NT - 1
    ext_extra = ext_extra + jnp.where(
        jnp.logical_and(is_last, iota8 == cnt), sent, 0.0)

    ext_tok = pltpu.roll(ext_tok, r, 0)
    ext_extra = pltpu.roll(ext_extra, r, 0)
    keep = iota8 < r
    old_tok = down_ref[pl.ds(a8, T + 8), :]
    old_extra = pp_ref[pl.ds(a8, T + 8), :]
    down_ref[pl.ds(a8, T + 8), :] = jnp.where(keep, old_tok, ext_tok)
    pp_ref[pl.ds(a8, T + 8), :] = jnp.where(keep, old_extra, ext_extra)
    base_ref[0] = base_prev + cnt

    sumg = jnp.where(t == 0, 0.0, sumg_ref[0]) + jnp.sum(probs)
    sumg_ref[0] = sumg

    @pl.when(is_last)
    def _finish():
        pltpu.make_async_copy(down_ref, down_hbm.at[b], sem1).start()
        pltpu.make_async_copy(pp_ref, pp_hbm.at[b], sem2).start()

        @pl.when(b == B - 1)
        def _final_drain():
            pltpu.make_async_copy(down_ref, down_hbm.at[b], sem1).wait()
            pltpu.make_async_copy(pp_ref, pp_hbm.at[b], sem2).wait()

        nb = base_prev + cnt
        F = nb.astype(jnp.float32) / SEQ
        G = sumg / SEQ
        auxb = (N_TGT / (N_TGT - 1.0)) * (
            (N_TGT - 1.0) * F * G + (1.0 - F) * (1.0 - G))
        contrib = auxb * (RATIO_W / B)
        prev = jnp.where(b == 0, 0.0, aux_ref[0, 0])
        aux_ref[0, 0] = prev + contrib


def _chunker(tokens, W_qk, start_key):
    return pl.pallas_call(
        _body,
        grid=(B, NT),
        in_specs=[
            pl.BlockSpec((1, T, D), lambda b, t: (b, t, 0)),
            pl.BlockSpec((D, 2 * D), lambda b, t: (0, 0)),
            pl.BlockSpec((1, D), lambda b, t: (0, 0)),
        ],
        out_specs=[
            pl.BlockSpec(memory_space=pl.ANY),
            pl.BlockSpec(memory_space=pl.ANY),
            pl.BlockSpec((1, 1), lambda b, t: (0, 0),
                         memory_space=pltpu.SMEM),
        ],
        out_shape=[
            jax.ShapeDtypeStruct((B, SEQ + 8, D), jnp.float32),
            jax.ShapeDtypeStruct((B, SEQ + PPW, PPW), jnp.float32),
            jax.ShapeDtypeStruct((1, 1), jnp.float32),
        ],
        scratch_shapes=[
            pltpu.VMEM((SEQ + 8, D), jnp.float32),
            pltpu.VMEM((SEQ + PPW, PPW), jnp.float32),
            pltpu.VMEM((1, D), jnp.float32),
            pltpu.SMEM((1,), jnp.int32),
            pltpu.SMEM((1,), jnp.float32),
            pltpu.SemaphoreType.DMA,
            pltpu.SemaphoreType.DMA,
        ],
        compiler_params=pltpu.CompilerParams(
            dimension_semantics=("arbitrary", "arbitrary")),
    )(tokens, W_qk, start_key)


def kernel(tokens, W_qk, start_key):
    down, pp, aux = _chunker(tokens, W_qk, start_key)
    down = down[:, :SEQ]
    probs_packed = pp[:, :SEQ, 0]
    sel = jnp.round(pp[:, :SEQ + 1, 1]).astype(jnp.int32)
    chunk_lens = jnp.maximum(sel[:, 1:] - sel[:, :-1], 0)
    gates = 1.0 - probs_packed
    return down, chunk_lens, gates, aux.reshape(())


# speedup vs baseline: 1.2596x; 1.2596x over previous
import jax
import jax.numpy as jnp
from jax.experimental import pallas as pl
from jax.experimental.pallas import tpu as pltpu

D = 768
SEQ = 8192
B = 4
T = 256
NT = SEQ // T
EPS = 1e-8
THR = 0.5
N_TGT = 6.0
RATIO_W = 0.03
PPW = 8


def _body(tok_ref, w_ref, sk_ref, down_hbm, pp_hbm, aux_ref,
          down_ref, pp_ref, carry_ref, base_ref, sumg_ref, sem1, sem2):
    b = pl.program_id(0)
    t = pl.program_id(1)

    x = tok_ref[0]
    qk = jnp.dot(x, w_ref[...], preferred_element_type=jnp.float32)
    q = qk[:, :D]
    k = qk[:, D:]

    carry = jnp.where(t == 0, sk_ref[...], carry_ref[...])
    kprev = jnp.concatenate([carry, k[:-1, :]], axis=0)
    carry_ref[...] = k[T - 1:T, :]

    dot = jnp.sum(q * kprev, axis=1, keepdims=True)
    qn = jnp.sqrt(jnp.sum(q * q, axis=1, keepdims=True))
    kn = jnp.sqrt(jnp.sum(kprev * kprev, axis=1, keepdims=True))
    den = jnp.maximum(qn, EPS) * jnp.maximum(kn, EPS)
    cos = dot / den
    probs = (1.0 - cos) * 0.5

    pos0 = t * T
    sub_iota = jax.lax.broadcasted_iota(jnp.int32, (T, 1), 0)
    boundary = jnp.logical_or(probs > THR, (sub_iota + pos0) == 0)
    bf = boundary.astype(jnp.float32)

    row_i = jax.lax.broadcasted_iota(jnp.int32, (T, T), 0)
    col_i = jax.lax.broadcasted_iota(jnp.int32, (T, T), 1)
    tri = (col_i < row_i).astype(jnp.float32)
    ranks_f = jax.lax.dot_general(tri, bf, (((1,), (0,)), ((), ())))
    ranks = ranks_f.astype(jnp.int32)
    cnt = jnp.sum(bf).astype(jnp.int32)

    P = jnp.where(jnp.logical_and(boundary, col_i == ranks), 1.0, 0.0)

    posf = (sub_iota + pos0).astype(jnp.float32)
    extra = jnp.concatenate(
        [probs, posf, jnp.zeros((T, PPW - 2), jnp.float32)], axis=1)
    G = jnp.concatenate([x * probs, extra], axis=1)

    Pb = P.astype(jnp.bfloat16)
    hi = G.astype(jnp.bfloat16)
    rem = G - hi.astype(jnp.float32)
    mid = rem.astype(jnp.bfloat16)
    lo = (rem - mid.astype(jnp.float32)).astype(jnp.bfloat16)
    dn = (((0,), (0,)), ((), ()))
    packed = (jax.lax.dot_general(Pb, hi, dn,
                                  preferred_element_type=jnp.float32)
              + jax.lax.dot_general(Pb, mid, dn,
                                    preferred_element_type=jnp.float32)
              + jax.lax.dot_general(Pb, lo, dn,
                                    preferred_element_type=jnp.float32))
    packed_tok = packed[:, :D]
    packed_extra = packed[:, D:]

    base_prev = jnp.where(t == 0, 0, base_ref[0])
    a8 = pl.multiple_of((base_prev // 8) * 8, 8)
    r = base_prev - a8

    @pl.when(jnp.logical_and(t == 0, b > 0))
    def _drain():
        pltpu.make_async_copy(down_ref, down_hbm.at[b - 1], sem1).wait()
        pltpu.make_async_copy(pp_ref, pp_hbm.at[b - 1], sem2).wait()

    down_ref[pl.ds(pos0, T), :] = jnp.zeros((T, D), jnp.float32)
    pp_ref[pl.ds(pos0, T), :] = jnp.zeros((T, PPW), jnp.float32)

    @pl.when(t == 0)
    def _zero_tail():
        pp_ref[pl.ds(SEQ, PPW), :] = jnp.zeros((PPW, PPW), jnp.float32)

    iota8 = jax.lax.broadcasted_iota(jnp.int32, (T + 8, 1), 0)
    ext_tok = jnp.concatenate(
        [packed_tok, jnp.zeros((8, D), jnp.float32)], axis=0)
    ext_extra = jnp.concatenate(
        [packed_extra, jnp.zeros((8, PPW), jnp.float32)], axis=0)
    sent = jnp.concatenate(
        [jnp.zeros((1, 1), jnp.float32),
         jnp.full((1, 1), float(SEQ), jnp.float32),
         jnp.zeros((1, PPW - 2), jnp.float32)], axis=1)
    is_last = t == NT - 1
    ext_extra = ext_extra + jnp.where(
        jnp.logical_and(is_last, iota8 == cnt), sent, 0.0)

    ext_tok = pltpu.roll(ext_tok, r, 0)
    ext_extra = pltpu.roll(ext_extra, r, 0)
    keep = iota8 < r
    old_tok = down_ref[pl.ds(a8, T + 8), :]
    old_extra = pp_ref[pl.ds(a8, T + 8), :]
    down_ref[pl.ds(a8, T + 8), :] = jnp.where(keep, old_tok, ext_tok)
    pp_ref[pl.ds(a8, T + 8), :] = jnp.where(keep, old_extra, ext_extra)
    base_ref[0] = base_prev + cnt

    sumg = jnp.where(t == 0, 0.0, sumg_ref[0]) + jnp.sum(probs)
    sumg_ref[0] = sumg

    @pl.when(is_last)
    def _finish():
        pltpu.make_async_copy(down_ref, down_hbm.at[b], sem1).start()
        pltpu.make_async_copy(pp_ref, pp_hbm.at[b], sem2).start()

        @pl.when(b == B - 1)
        def _final_drain():
            pltpu.make_async_copy(down_ref, down_hbm.at[b], sem1).wait()
            pltpu.make_async_copy(pp_ref, pp_hbm.at[b], sem2).wait()

        nb = base_prev + cnt
        F = nb.astype(jnp.float32) / SEQ
        G = sumg / SEQ
        auxb = (N_TGT / (N_TGT - 1.0)) * (
            (N_TGT - 1.0) * F * G + (1.0 - F) * (1.0 - G))
        contrib = auxb * (RATIO_W / B)
        prev = jnp.where(b == 0, 0.0, aux_ref[0, 0])
        aux_ref[0, 0] = prev + contrib


def _chunker(tokens, W_qk, start_key):
    return pl.pallas_call(
        _body,
        grid=(B, NT),
        in_specs=[
            pl.BlockSpec((1, T, D), lambda b, t: (b, t, 0)),
            pl.BlockSpec((D, 2 * D), lambda b, t: (0, 0)),
            pl.BlockSpec((1, D), lambda b, t: (0, 0)),
        ],
        out_specs=[
            pl.BlockSpec(memory_space=pl.ANY),
            pl.BlockSpec(memory_space=pl.ANY),
            pl.BlockSpec((1, 1), lambda b, t: (0, 0),
                         memory_space=pltpu.SMEM),
        ],
        out_shape=[
            jax.ShapeDtypeStruct((B, SEQ + 8, D), jnp.float32),
            jax.ShapeDtypeStruct((B, SEQ + PPW, PPW), jnp.float32),
            jax.ShapeDtypeStruct((1, 1), jnp.float32),
        ],
        scratch_shapes=[
            pltpu.VMEM((SEQ + 8, D), jnp.float32),
            pltpu.VMEM((SEQ + PPW, PPW), jnp.float32),
            pltpu.VMEM((1, D), jnp.float32),
            pltpu.SMEM((1,), jnp.int32),
            pltpu.SMEM((1,), jnp.float32),
            pltpu.SemaphoreType.DMA,
            pltpu.SemaphoreType.DMA,
        ],
        compiler_params=pltpu.CompilerParams(
            dimension_semantics=("arbitrary", "arbitrary")),
    )(tokens, W_qk, start_key)


def kernel(tokens, W_qk, start_key):
    down, pp, aux = _chunker(tokens, W_qk, start_key)
    down = down[:, :SEQ]
    probs_packed = pp[:, :SEQ, 0]
    sel = jnp.round(pp[:, :SEQ + 1, 1]).astype(jnp.int32)
    chunk_lens = jnp.maximum(sel[:, 1:] - sel[:, :-1], 0)
    gates = 1.0 - probs_packed
    return down, chunk_lens, gates, aux.reshape(())


# speedup vs baseline: 1.4840x; 1.1782x over previous
import jax
import jax.numpy as jnp
from jax.experimental import pallas as pl
from jax.experimental.pallas import tpu as pltpu

D = 768
SEQ = 8192
B = 4
T = 256
NT = SEQ // T
EPS = 1e-8
THR = 0.5
N_TGT = 6.0
RATIO_W = 0.03
PPW = 8


def _body(tok_ref, w_ref, sk_ref, down_hbm, pp_hbm, aux_ref,
          down_ref, pp_ref, carry_ref, base_ref, sumg_ref, sem1, sem2):
    b = pl.program_id(0)
    t = pl.program_id(1)

    x = tok_ref[0]
    qk = jnp.dot(x, w_ref[...], preferred_element_type=jnp.float32)
    q = qk[:, :D]
    k = qk[:, D:]

    carry = jnp.where(t == 0, sk_ref[...], carry_ref[...])
    kprev = jnp.concatenate([carry, k[:-1, :]], axis=0)
    carry_ref[...] = k[T - 1:T, :]

    dot = jnp.sum(q * kprev, axis=1, keepdims=True)
    qn = jnp.sqrt(jnp.sum(q * q, axis=1, keepdims=True))
    kn = jnp.sqrt(jnp.sum(kprev * kprev, axis=1, keepdims=True))
    den = jnp.maximum(qn, EPS) * jnp.maximum(kn, EPS)
    cos = dot / den
    probs = (1.0 - cos) * 0.5

    pos0 = t * T
    sub_iota = jax.lax.broadcasted_iota(jnp.int32, (T, 1), 0)
    boundary = jnp.logical_or(probs > THR, (sub_iota + pos0) == 0)
    bf = boundary.astype(jnp.float32)

    row_i = jax.lax.broadcasted_iota(jnp.int32, (T, T), 0)
    col_i = jax.lax.broadcasted_iota(jnp.int32, (T, T), 1)
    tri = (col_i < row_i).astype(jnp.float32)
    ranks_f = jax.lax.dot_general(tri, bf, (((1,), (0,)), ((), ())))
    ranks = ranks_f.astype(jnp.int32)
    cnt = jnp.sum(bf).astype(jnp.int32)

    base_prev = jnp.where(t == 0, 0, base_ref[0])
    a8 = pl.multiple_of((base_prev // 8) * 8, 8)
    r = base_prev - a8

    col_i8 = jax.lax.broadcasted_iota(jnp.int32, (T, T + 8), 1)
    P = jnp.where(jnp.logical_and(boundary, col_i8 == ranks + r), 1.0, 0.0)

    posf = (sub_iota + pos0).astype(jnp.float32)
    extra = jnp.concatenate(
        [probs, posf, jnp.zeros((T, PPW - 2), jnp.float32)], axis=1)
    G = jnp.concatenate([x * probs, extra], axis=1)

    Pb = P.astype(jnp.bfloat16)
    hi = G.astype(jnp.bfloat16)
    mid = (G - hi.astype(jnp.float32)).astype(jnp.bfloat16)
    dn = (((0,), (0,)), ((), ()))
    packed = (jax.lax.dot_general(Pb, hi, dn,
                                  preferred_element_type=jnp.float32)
              + jax.lax.dot_general(Pb, mid, dn,
                                    preferred_element_type=jnp.float32))
    ext_tok = packed[:, :D]
    ext_extra = packed[:, D:]

    @pl.when(jnp.logical_and(t == 0, b > 0))
    def _drain():
        pltpu.make_async_copy(down_ref, down_hbm.at[b - 1], sem1).wait()
        pltpu.make_async_copy(pp_ref, pp_hbm.at[b - 1], sem2).wait()

    down_ref[pl.ds(pos0, T), :] = jnp.zeros((T, D), jnp.float32)
    pp_ref[pl.ds(pos0, T), :] = jnp.zeros((T, PPW), jnp.float32)

    @pl.when(t == 0)
    def _zero_tail():
        pp_ref[pl.ds(SEQ, PPW), :] = jnp.zeros((PPW, PPW), jnp.float32)

    iota8 = jax.lax.broadcasted_iota(jnp.int32, (T + 8, 1), 0)
    sent = jnp.concatenate(
        [jnp.zeros((1, 1), jnp.float32),
         jnp.full((1, 1), float(SEQ), jnp.float32),
         jnp.zeros((1, PPW - 2), jnp.float32)], axis=1)
    is_last = t == NT - 1
    ext_extra = ext_extra + jnp.where(
        jnp.logical_and(is_last, iota8 == cnt + r), sent, 0.0)

    keep = iota8 < r
    old_tok = down_ref[pl.ds(a8, T + 8), :]
    old_extra = pp_ref[pl.ds(a8, T + 8), :]
    down_ref[pl.ds(a8, T + 8), :] = jnp.where(keep, old_tok, ext_tok)
    pp_ref[pl.ds(a8, T + 8), :] = jnp.where(keep, old_extra, ext_extra)
    base_ref[0] = base_prev + cnt

    sumg = jnp.where(t == 0, 0.0, sumg_ref[0]) + jnp.sum(probs)
    sumg_ref[0] = sumg

    @pl.when(is_last)
    def _finish():
        pltpu.make_async_copy(down_ref, down_hbm.at[b], sem1).start()
        pltpu.make_async_copy(pp_ref, pp_hbm.at[b], sem2).start()

        @pl.when(b == B - 1)
        def _final_drain():
            pltpu.make_async_copy(down_ref, down_hbm.at[b], sem1).wait()
            pltpu.make_async_copy(pp_ref, pp_hbm.at[b], sem2).wait()

        nb = base_prev + cnt
        F = nb.astype(jnp.float32) / SEQ
        G = sumg / SEQ
        auxb = (N_TGT / (N_TGT - 1.0)) * (
            (N_TGT - 1.0) * F * G + (1.0 - F) * (1.0 - G))
        contrib = auxb * (RATIO_W / B)
        prev = jnp.where(b == 0, 0.0, aux_ref[0, 0])
        aux_ref[0, 0] = prev + contrib


def _chunker(tokens, W_qk, start_key):
    return pl.pallas_call(
        _body,
        grid=(B, NT),
        in_specs=[
            pl.BlockSpec((1, T, D), lambda b, t: (b, t, 0)),
            pl.BlockSpec((D, 2 * D), lambda b, t: (0, 0)),
            pl.BlockSpec((1, D), lambda b, t: (0, 0)),
        ],
        out_specs=[
            pl.BlockSpec(memory_space=pl.ANY),
            pl.BlockSpec(memory_space=pl.ANY),
            pl.BlockSpec((1, 1), lambda b, t: (0, 0),
                         memory_space=pltpu.SMEM),
        ],
        out_shape=[
            jax.ShapeDtypeStruct((B, SEQ + 8, D), jnp.float32),
            jax.ShapeDtypeStruct((B, SEQ + PPW, PPW), jnp.float32),
            jax.ShapeDtypeStruct((1, 1), jnp.float32),
        ],
        scratch_shapes=[
            pltpu.VMEM((SEQ + 8, D), jnp.float32),
            pltpu.VMEM((SEQ + PPW, PPW), jnp.float32),
            pltpu.VMEM((1, D), jnp.float32),
            pltpu.SMEM((1,), jnp.int32),
            pltpu.SMEM((1,), jnp.float32),
            pltpu.SemaphoreType.DMA,
            pltpu.SemaphoreType.DMA,
        ],
        compiler_params=pltpu.CompilerParams(
            dimension_semantics=("arbitrary", "arbitrary")),
    )(tokens, W_qk, start_key)


def kernel(tokens, W_qk, start_key):
    down, pp, aux = _chunker(tokens, W_qk, start_key)
    down = down[:, :SEQ]
    probs_packed = pp[:, :SEQ, 0]
    sel = jnp.round(pp[:, :SEQ + 1, 1]).astype(jnp.int32)
    chunk_lens = jnp.maximum(sel[:, 1:] - sel[:, :-1], 0)
    gates = 1.0 - probs_packed
    return down, chunk_lens, gates, aux.reshape(())


# speedup vs baseline: 1.7533x; 1.1815x over previous
import jax
import jax.numpy as jnp
from jax.experimental import pallas as pl
from jax.experimental.pallas import tpu as pltpu

D = 768
SEQ = 8192
B = 4
T = 256
NT = SEQ // T
EPS = 1e-8
THR = 0.5
N_TGT = 6.0
RATIO_W = 0.03
PPW = 8


def _body(tok_ref, w_ref, sk_ref, down_hbm, pp_hbm, aux_ref,
          down_ref, pp_ref, carry_ref, base_ref, sumg_ref, sem1, sem2):
    b = pl.program_id(0)
    t = pl.program_id(1)

    x = tok_ref[0]
    qk = jnp.dot(x, w_ref[...], preferred_element_type=jnp.float32)
    q = qk[:, :D]
    k = qk[:, D:]

    carry = jnp.where(t == 0, sk_ref[...], carry_ref[...])
    kprev = jnp.concatenate([carry, k[:-1, :]], axis=0)
    carry_ref[...] = k[T - 1:T, :]

    dot = jnp.sum(q * kprev, axis=1, keepdims=True)
    qn = jnp.sqrt(jnp.sum(q * q, axis=1, keepdims=True))
    kn = jnp.sqrt(jnp.sum(kprev * kprev, axis=1, keepdims=True))
    den = jnp.maximum(qn, EPS) * jnp.maximum(kn, EPS)
    cos = dot / den
    probs = (1.0 - cos) * 0.5

    pos0 = t * T
    sub_iota = jax.lax.broadcasted_iota(jnp.int32, (T, 1), 0)
    boundary = jnp.logical_or(probs > THR, (sub_iota + pos0) == 0)
    bf = boundary.astype(jnp.float32)

    row_i = jax.lax.broadcasted_iota(jnp.int32, (T, T), 0)
    col_i = jax.lax.broadcasted_iota(jnp.int32, (T, T), 1)
    tri = (col_i < row_i).astype(jnp.float32)
    ranks_f = jax.lax.dot_general(tri, bf, (((1,), (0,)), ((), ())))
    ranks = ranks_f.astype(jnp.int32)
    cnt = jnp.sum(bf).astype(jnp.int32)

    base_prev = jnp.where(t == 0, 0, base_ref[0])
    a8 = pl.multiple_of((base_prev // 8) * 8, 8)
    r = base_prev - a8

    col_i8 = jax.lax.broadcasted_iota(jnp.int32, (T, T + 8), 1)
    P = jnp.where(jnp.logical_and(boundary, col_i8 == ranks + r), 1.0, 0.0)

    posf = (sub_iota + pos0).astype(jnp.float32)
    extra = jnp.concatenate(
        [probs, posf, jnp.zeros((T, PPW - 2), jnp.float32)], axis=1)
    G = jnp.concatenate([x * probs, extra], axis=1)

    Pb = P.astype(jnp.bfloat16)
    hi = G.astype(jnp.bfloat16)
    mid = (G - hi.astype(jnp.float32)).astype(jnp.bfloat16)
    dn = (((0,), (0,)), ((), ()))
    packed = (jax.lax.dot_general(Pb, hi, dn,
                                  preferred_element_type=jnp.float32)
              + jax.lax.dot_general(Pb, mid, dn,
                                    preferred_element_type=jnp.float32))
    ext_tok = packed[:, :D]
    ext_extra = packed[:, D:]

    @pl.when(jnp.logical_and(t == 0, b > 0))
    def _drain():
        pltpu.make_async_copy(down_ref, down_hbm.at[b - 1], sem1).wait()
        pltpu.make_async_copy(pp_ref, pp_hbm.at[b - 1], sem2).wait()

    down_ref[pl.ds(pos0, T), :] = jnp.zeros((T, D), jnp.float32)
    pp_ref[pl.ds(pos0, T), :] = jnp.zeros((T, PPW), jnp.float32)

    @pl.when(t == 0)
    def _zero_tail():
        pp_ref[pl.ds(SEQ, PPW), :] = jnp.zeros((PPW, PPW), jnp.float32)

    iota8 = jax.lax.broadcasted_iota(jnp.int32, (T + 8, 1), 0)
    sent = jnp.concatenate(
        [jnp.zeros((1, 1), jnp.float32),
         jnp.full((1, 1), float(SEQ), jnp.float32),
         jnp.zeros((1, PPW - 2), jnp.float32)], axis=1)
    is_last = t == NT - 1
    ext_extra = ext_extra + jnp.where(
        jnp.logical_and(is_last, iota8 == cnt + r), sent, 0.0)

    keep = iota8 < r
    old_tok = down_ref[pl.ds(a8, T), :]
    old_extra = pp_ref[pl.ds(a8, T + 8), :]
    down_ref[pl.ds(a8, T), :] = jnp.where(keep[:T], old_tok, ext_tok[:T])
    pp_ref[pl.ds(a8, T + 8), :] = jnp.where(keep, old_extra, ext_extra)

    @pl.when(a8 + T < SEQ)
    def _spill():
        down_ref[pl.ds(a8 + T, 8), :] = ext_tok[T:, :]

    base_ref[0] = base_prev + cnt

    sumg = jnp.where(t == 0, 0.0, sumg_ref[0]) + jnp.sum(probs)
    sumg_ref[0] = sumg

    @pl.when(is_last)
    def _finish():
        pltpu.make_async_copy(down_ref, down_hbm.at[b], sem1).start()
        pltpu.make_async_copy(pp_ref, pp_hbm.at[b], sem2).start()

        @pl.when(b == B - 1)
        def _final_drain():
            pltpu.make_async_copy(down_ref, down_hbm.at[b], sem1).wait()
            pltpu.make_async_copy(pp_ref, pp_hbm.at[b], sem2).wait()

        nb = base_prev + cnt
        F = nb.astype(jnp.float32) / SEQ
        G = sumg / SEQ
        auxb = (N_TGT / (N_TGT - 1.0)) * (
            (N_TGT - 1.0) * F * G + (1.0 - F) * (1.0 - G))
        contrib = auxb * (RATIO_W / B)
        prev = jnp.where(b == 0, 0.0, aux_ref[0, 0])
        aux_ref[0, 0] = prev + contrib


def _chunker(tokens, W_qk, start_key):
    return pl.pallas_call(
        _body,
        grid=(B, NT),
        in_specs=[
            pl.BlockSpec((1, T, D), lambda b, t: (b, t, 0)),
            pl.BlockSpec((D, 2 * D), lambda b, t: (0, 0)),
            pl.BlockSpec((1, D), lambda b, t: (0, 0)),
        ],
        out_specs=[
            pl.BlockSpec(memory_space=pl.ANY),
            pl.BlockSpec(memory_space=pl.ANY),
            pl.BlockSpec((1, 1), lambda b, t: (0, 0),
                         memory_space=pltpu.SMEM),
        ],
        out_shape=[
            jax.ShapeDtypeStruct((B, SEQ, D), jnp.float32),
            jax.ShapeDtypeStruct((B, SEQ + PPW, PPW), jnp.float32),
            jax.ShapeDtypeStruct((1, 1), jnp.float32),
        ],
        scratch_shapes=[
            pltpu.VMEM((SEQ, D), jnp.float32),
            pltpu.VMEM((SEQ + PPW, PPW), jnp.float32),
            pltpu.VMEM((1, D), jnp.float32),
            pltpu.SMEM((1,), jnp.int32),
            pltpu.SMEM((1,), jnp.float32),
            pltpu.SemaphoreType.DMA,
            pltpu.SemaphoreType.DMA,
        ],
        compiler_params=pltpu.CompilerParams(
            dimension_semantics=("arbitrary", "arbitrary")),
    )(tokens, W_qk, start_key)


def kernel(tokens, W_qk, start_key):
    down, pp, aux = _chunker(tokens, W_qk, start_key)
    probs_packed = pp[:, :SEQ, 0]
    sel = jnp.round(pp[:, :SEQ + 1, 1]).astype(jnp.int32)
    chunk_lens = jnp.maximum(sel[:, 1:] - sel[:, :-1], 0)
    gates = 1.0 - probs_packed
    return down, chunk_lens, gates, aux.reshape(())
